# Initial kernel scaffold; baseline (speedup 1.0000x reference)
import jax
import jax.numpy as jnp
from jax.experimental import pallas as pl


def kernel(x, table):
    raise NotImplementedError("write your pallas kernel here")


# speedup vs baseline: 2.4106x; 2.4106x over previous
import math

import jax
import jax.numpy as jnp
from jax import lax
from jax.experimental import pallas as pl
from jax.experimental.pallas import tpu as pltpu
from jax.experimental.pallas import tpu_sc as plsc

D_MODEL = 128
SCALE = math.sqrt(D_MODEL)
NC, NS, LANES = 2, 16, 16
NW = NC * NS
CHUNK = 128


def _body(x3d_hbm, table_hbm, out_hbm, idx_v, rows_v, sem):
    n_chunks = x3d_hbm.shape[1]
    wid = lax.axis_index("s") * NC + lax.axis_index("c")
    base = wid * n_chunks
    pltpu.sync_copy(x3d_hbm.at[wid], idx_v)

    def chunk_body(c, carry):
        pltpu.async_copy(table_hbm.at[idx_v.at[c]], rows_v, sem).wait()

        def scale_row(r, carry2):
            for j in range(D_MODEL // LANES):
                sl = pl.ds(j * LANES, LANES)
                rows_v[r, sl] = rows_v[r, sl] * SCALE
            return carry2

        lax.fori_loop(0, CHUNK, scale_row, 0, unroll=2)
        pltpu.sync_copy(rows_v, out_hbm.at[pl.ds((base + c) * CHUNK, CHUNK)])
        return carry

    lax.fori_loop(0, n_chunks, chunk_body, 0)


def kernel(x, table):
    orig_shape = x.shape
    b_total = x.size
    assert b_total % (NW * CHUNK) == 0
    n_chunks = b_total // (NW * CHUNK)
    x3d = x.reshape(NW, n_chunks, CHUNK).astype(jnp.int32)

    mesh = plsc.VectorSubcoreMesh(core_axis_name="c", subcore_axis_name="s")
    out = pl.kernel(
        _body,
        out_type=jax.ShapeDtypeStruct((b_total, D_MODEL), jnp.float32),
        mesh=mesh,
        scratch_types=[
            pltpu.VMEM((n_chunks, CHUNK), jnp.int32),
            pltpu.VMEM((CHUNK, D_MODEL), jnp.float32),
            pltpu.SemaphoreType.DMA,
        ],
    )(x3d, table)
    return out.reshape(*orig_shape, D_MODEL)
